# baseline (speedup 1.0000x reference)
import functools

import jax
import jax.numpy as jnp
from jax import lax
from jax.experimental import pallas as pl
from jax.experimental.pallas import tpu as pltpu
from jax.experimental.pallas import tpu_sc as plsc

VOCAB = 1000000
D_EMBED = 32
BATCH = 16384
HIST = 50

NC = 2
NS = 16
NW = NC * NS

RPT = VOCAB // NS
TCH = 625
NTCH = RPT // TCH

BCH = 512
NBCH = BATCH // BCH
SUB = 128
KSUB = BCH // SUB
NCHT = HIST * NBCH // NW
NCH2 = (HIST - NW) * NBCH // NW

_MESH = plsc.VectorSubcoreMesh(core_axis_name="c", subcore_axis_name="s")


def _acc_rows(buf, n, acc, unroll=8):
    def row(i, c):
        c0, c1 = c
        return (c0 + buf[i, pl.ds(0, 16)], c1 + buf[i, pl.ds(16, 16)])

    return lax.fori_loop(0, n, row, acc, unroll=unroll)


@functools.partial(
    pl.kernel,
    mesh=_MESH,
    compiler_params=pltpu.CompilerParams(
        use_tc_tiling_on_sc=False, needs_layout_passes=False),
    out_type=jax.ShapeDtypeStruct((HIST, D_EMBED, BATCH), jnp.float32),
    scratch_types=[
        pltpu.VMEM((TCH, D_EMBED), jnp.float32),
        pltpu.VMEM((TCH, D_EMBED), jnp.float32),
        pltpu.VMEM((32,), jnp.float32),
        pltpu.VMEM((NS, 32), jnp.float32),
        pltpu.VMEM_SHARED((NS, 32), jnp.float32),
        pltpu.VMEM((KSUB, SUB), jnp.int32),
        pltpu.VMEM((KSUB, SUB), jnp.int32),
        pltpu.VMEM((BCH, D_EMBED), jnp.float32),
        pltpu.VMEM((BCH, D_EMBED), jnp.float32),
        pltpu.VMEM((D_EMBED, BCH + 1), jnp.float32),
        pltpu.SemaphoreType.DMA,
        pltpu.SemaphoreType.DMA,
    ],
)
def _zme(x_hbm, table_hbm, out_hbm,
         tbuf0, tbuf1, part_v, all_v, shared, idx_a, idx_b, rows_a, rows_b,
         trows_v, sem1, sem2):
    sid = lax.axis_index("s")
    wid = sid * NC + lax.axis_index("c")

    row0 = sid * RPT
    pltpu.async_copy(table_hbm.at[pl.ds(row0, TCH)], tbuf0, sem1)
    zero = jnp.zeros((16,), jnp.float32)

    def two(g, acc):
        pltpu.async_copy(table_hbm.at[pl.ds(row0 + (2 * g + 1) * TCH, TCH)],
                         tbuf1, sem1)
        pltpu.make_async_copy(table_hbm.at[pl.ds(row0, TCH)], tbuf0, sem1).wait()
        acc = _acc_rows(tbuf0, TCH, acc)

        @pl.when(g < NTCH // 2 - 1)
        def _():
            pltpu.async_copy(table_hbm.at[pl.ds(row0 + (2 * g + 2) * TCH, TCH)],
                             tbuf0, sem1)

        pltpu.make_async_copy(table_hbm.at[pl.ds(row0, TCH)], tbuf1, sem1).wait()
        return _acc_rows(tbuf1, TCH, acc)

    a0, a1 = lax.fori_loop(0, NTCH // 2, two, (zero, zero))
    part_v[pl.ds(0, 16)] = a0
    part_v[pl.ds(16, 16)] = a1

    pltpu.sync_copy(part_v, shared.at[sid])
    plsc.subcore_barrier()
    pltpu.sync_copy(shared, all_v)

    def fold(i, c):
        c0, c1 = c
        return (c0 + all_v[i, pl.ds(0, 16)], c1 + all_v[i, pl.ds(16, 16)])

    scale = -1.0 / VOCAB
    m0, m1 = lax.fori_loop(0, NS, fold, (zero, zero), unroll=4)
    nm0 = m0 * scale
    nm1 = m1 * scale

    lanes = lax.iota(jnp.int32, 16)

    def locate(k):
        cid = wid * NCH2 + (k - NBCH)
        h = jnp.where(k < NBCH, wid, NW + cid // NBCH)
        bg = jnp.where(k < NBCH, k, cid % NBCH)
        return h, bg

    def fire(k, idx_v, rows_v):
        h, bg = locate(k)
        pltpu.sync_copy(x_hbm.at[h, pl.ds(bg * KSUB, KSUB)], idx_v)
        for j in range(KSUB):
            pltpu.async_copy(
                table_hbm.at[idx_v.at[j]],
                rows_v.at[pl.ds(j * SUB, SUB)],
                sem2,
            )

    def process(k, rows_v):
        for j in range(KSUB):
            pltpu.make_async_copy(
                table_hbm.at[idx_a.at[j]],
                rows_v.at[pl.ds(j * SUB, SUB)],
                sem2,
            ).wait()

        def row(i, col):
            plsc.store_scatter(trows_v, [lanes, col],
                               rows_v[i, pl.ds(0, 16)] + nm0)
            plsc.store_scatter(trows_v, [lanes + 16, col],
                               rows_v[i, pl.ds(16, 16)] + nm1)
            return col + 1

        lax.fori_loop(0, BCH, row, jnp.zeros((16,), jnp.int32), unroll=8)
        h, bg = locate(k)
        pltpu.sync_copy(trows_v.at[:, pl.ds(0, BCH)],
                        out_hbm.at[h, :, pl.ds(bg * BCH, BCH)])

    fire(0, idx_a, rows_a)

    def pipelined(p, carry):
        k0 = 2 * p
        fire(k0 + 1, idx_b, rows_b)
        process(k0, rows_a)

        @pl.when(p < NCHT // 2 - 1)
        def _():
            fire(k0 + 2, idx_a, rows_a)

        process(k0 + 1, rows_b)
        return carry

    lax.fori_loop(0, NCHT // 2, pipelined, 0)


def kernel(x, weight):
    xt3 = x.T.reshape(HIST, BATCH // SUB, SUB).astype(jnp.int32)
    out = _zme(xt3, weight)
    return jnp.transpose(out, (2, 0, 1))
